# Initial kernel scaffold; baseline (speedup 1.0000x reference)
import jax
import jax.numpy as jnp
from jax.experimental import pallas as pl


def kernel(x, Wq, mem_keys, mem_values, Wout, Wg1, bg1, Wg2, bg2):
    raise NotImplementedError("write your pallas kernel here")


# speedup vs baseline: 28.1718x; 28.1718x over previous
import functools
import math

import jax
import jax.numpy as jnp
from jax import lax
from jax.experimental import pallas as pl
from jax.experimental.pallas import tpu as pltpu

B, T, D, M, K = 2, 1024, 128, 32768, 8
N = B * T
TT = 256
TM = 2048
NT = N // TT
NM = M // TM
INV_SQRT_D = 1.0 / math.sqrt(D)
NEG_INF = float("-inf")


def _topk_kernel(x_ref, wq_ref, keys_ref, vals_ref, idx_ref, q_scr):
    m = pl.program_id(1)

    @pl.when(m == 0)
    def _init():
        q_scr[...] = jnp.dot(x_ref[...], wq_ref[...].T,
                             preferred_element_type=jnp.float32)
        vals_ref[...] = jnp.full((TT, K), NEG_INF, jnp.float32)
        idx_ref[...] = jnp.zeros((TT, K), jnp.int32)

    sim = jnp.dot(q_scr[...], keys_ref[...].T,
                  preferred_element_type=jnp.float32) * INV_SQRT_D

    iota = lax.broadcasted_iota(jnp.int32, (TT, TM), 1)
    tvals = []
    tidx = []
    s = sim
    for _ in range(K):
        mx = jnp.max(s, axis=1, keepdims=True)
        pos = jnp.min(jnp.where(s == mx, iota, TM), axis=1, keepdims=True)
        tvals.append(mx)
        tidx.append(pos + m * TM)
        s = jnp.where(iota == pos, NEG_INF, s)

    cat_v = jnp.concatenate([vals_ref[...]] + tvals, axis=1)
    cat_i = jnp.concatenate([idx_ref[...]] + tidx, axis=1)
    iota2 = lax.broadcasted_iota(jnp.int32, (TT, 2 * K), 1)
    nv = []
    ni = []
    for _ in range(K):
        mx = jnp.max(cat_v, axis=1, keepdims=True)
        pos = jnp.min(jnp.where(cat_v == mx, iota2, 2 * K), axis=1,
                      keepdims=True)
        sel = jnp.min(jnp.where(iota2 == pos, cat_i, M), axis=1,
                      keepdims=True)
        nv.append(mx)
        ni.append(sel)
        cat_v = jnp.where(iota2 == pos, NEG_INF, cat_v)
    vals_ref[...] = jnp.concatenate(nv, axis=1)
    idx_ref[...] = jnp.concatenate(ni, axis=1)


def _topk(xf, Wq, mem_keys):
    return pl.pallas_call(
        _topk_kernel,
        grid=(NT, NM),
        in_specs=[
            pl.BlockSpec((TT, D), lambda t, m: (t, 0)),
            pl.BlockSpec((D, D), lambda t, m: (0, 0)),
            pl.BlockSpec((TM, D), lambda t, m: (m, 0)),
        ],
        out_specs=[
            pl.BlockSpec((TT, K), lambda t, m: (t, 0)),
            pl.BlockSpec((TT, K), lambda t, m: (t, 0)),
        ],
        out_shape=[
            jax.ShapeDtypeStruct((N, K), jnp.float32),
            jax.ShapeDtypeStruct((N, K), jnp.int32),
        ],
        scratch_shapes=[pltpu.VMEM((TT, D), jnp.float32)],
        compiler_params=pltpu.CompilerParams(
            dimension_semantics=("parallel", "arbitrary")),
    )(xf, Wq, mem_keys)


def _epilogue_kernel(x_ref, vals_ref, g_ref, wout_ref, wg1_ref, bg1_ref,
                     wg2_ref, bg2_ref, out_ref):
    vals = vals_ref[...]
    mx = jnp.max(vals, axis=1, keepdims=True)
    e = jnp.exp(vals - mx)
    w = e / jnp.sum(e, axis=1, keepdims=True)

    row_id = lax.broadcasted_iota(jnp.int32, (K, D), 0)
    g = g_ref[...]
    r = jnp.zeros((TT, D), jnp.float32)
    for k in range(K):
        sel = jnp.where(row_id == k, 1.0, 0.0)
        wb = jnp.dot(w, sel, preferred_element_type=jnp.float32)
        r = r + wb * g[:, k, :]

    r = jnp.dot(r, wout_ref[...].T, preferred_element_type=jnp.float32)
    x = x_ref[...]
    gate_in = jnp.concatenate([x, r], axis=1)
    h = jnp.dot(gate_in, wg1_ref[...].T,
                preferred_element_type=jnp.float32) + bg1_ref[...]
    h = 0.5 * h * (1.0 + lax.erf(h * (1.0 / math.sqrt(2.0))))
    gate = jnp.dot(h, wg2_ref[...].T,
                   preferred_element_type=jnp.float32) + bg2_ref[...]
    gate = jax.nn.sigmoid(gate)
    out_ref[...] = x + gate * r


def _epilogue(xf, vals, gathered, Wout, Wg1, bg1, Wg2, bg2):
    H = Wg1.shape[0]
    return pl.pallas_call(
        _epilogue_kernel,
        grid=(NT,),
        in_specs=[
            pl.BlockSpec((TT, D), lambda t: (t, 0)),
            pl.BlockSpec((TT, K), lambda t: (t, 0)),
            pl.BlockSpec((TT, K, D), lambda t: (t, 0, 0)),
            pl.BlockSpec((D, D), lambda t: (0, 0)),
            pl.BlockSpec((H, 2 * D), lambda t: (0, 0)),
            pl.BlockSpec((1, H), lambda t: (0, 0)),
            pl.BlockSpec((D, H), lambda t: (0, 0)),
            pl.BlockSpec((1, D), lambda t: (0, 0)),
        ],
        out_specs=pl.BlockSpec((TT, D), lambda t: (t, 0)),
        out_shape=jax.ShapeDtypeStruct((N, D), jnp.float32),
        compiler_params=pltpu.CompilerParams(
            dimension_semantics=("parallel",)),
    )(xf, vals, gathered, Wout, Wg1, bg1.reshape(1, H),
      jnp.broadcast_to(Wg2, (D, H)), jnp.broadcast_to(bg2.reshape(1, 1),
                                                      (1, D)))


@jax.jit
def kernel(x, Wq, mem_keys, mem_values, Wout, Wg1, bg1, Wg2, bg2):
    xf = x.reshape(N, D)
    vals, idx = _topk(xf, Wq, mem_keys)
    gathered = jnp.take(mem_values, idx, axis=0)
    out = _epilogue(xf, vals, gathered, Wout, Wg1, bg1, Wg2, bg2)
    return out.reshape(B, T, D)


# speedup vs baseline: 28.6001x; 1.0152x over previous
import functools
import math

import jax
import jax.numpy as jnp
from jax import lax
from jax.experimental import pallas as pl
from jax.experimental.pallas import tpu as pltpu
from jax.experimental.pallas import tpu_sc as plsc

B, T, D, M, K = 2, 1024, 128, 32768, 8
N = B * T
TT = 256
TM = 2048
NT = N // TT
NM = M // TM
INV_SQRT_D = 1.0 / math.sqrt(D)
NEG_INF = float("-inf")


def _topk_kernel(x_ref, wq_ref, keys_ref, vals_ref, idx_ref, q_scr):
    m = pl.program_id(1)

    @pl.when(m == 0)
    def _init():
        q_scr[...] = jnp.dot(x_ref[...], wq_ref[...].T,
                             preferred_element_type=jnp.float32)
        vals_ref[...] = jnp.full((TT, K), NEG_INF, jnp.float32)
        idx_ref[...] = jnp.zeros((TT, K), jnp.int32)

    sim = jnp.dot(q_scr[...], keys_ref[...].T,
                  preferred_element_type=jnp.float32) * INV_SQRT_D

    iota = lax.broadcasted_iota(jnp.int32, (TT, TM), 1)
    tvals = []
    tidx = []
    s = sim
    for _ in range(K):
        mx = jnp.max(s, axis=1, keepdims=True)
        pos = jnp.min(jnp.where(s == mx, iota, TM), axis=1, keepdims=True)
        tvals.append(mx)
        tidx.append(pos + m * TM)
        s = jnp.where(iota == pos, NEG_INF, s)

    cat_v = jnp.concatenate([vals_ref[...]] + tvals, axis=1)
    cat_i = jnp.concatenate([idx_ref[...]] + tidx, axis=1)
    iota2 = lax.broadcasted_iota(jnp.int32, (TT, 2 * K), 1)
    nv = []
    ni = []
    for _ in range(K):
        mx = jnp.max(cat_v, axis=1, keepdims=True)
        pos = jnp.min(jnp.where(cat_v == mx, iota2, 2 * K), axis=1,
                      keepdims=True)
        sel = jnp.min(jnp.where(iota2 == pos, cat_i, M), axis=1,
                      keepdims=True)
        nv.append(mx)
        ni.append(sel)
        cat_v = jnp.where(iota2 == pos, NEG_INF, cat_v)
    vals_ref[...] = jnp.concatenate(nv, axis=1)
    idx_ref[...] = jnp.concatenate(ni, axis=1)


def _topk(xf, Wq, mem_keys):
    return pl.pallas_call(
        _topk_kernel,
        grid=(NT, NM),
        in_specs=[
            pl.BlockSpec((TT, D), lambda t, m: (t, 0)),
            pl.BlockSpec((D, D), lambda t, m: (0, 0)),
            pl.BlockSpec((TM, D), lambda t, m: (m, 0)),
        ],
        out_specs=[
            pl.BlockSpec((TT, K), lambda t, m: (t, 0)),
            pl.BlockSpec((TT, K), lambda t, m: (t, 0)),
        ],
        out_shape=[
            jax.ShapeDtypeStruct((N, K), jnp.float32),
            jax.ShapeDtypeStruct((N, K), jnp.int32),
        ],
        scratch_shapes=[pltpu.VMEM((TT, D), jnp.float32)],
        compiler_params=pltpu.CompilerParams(
            dimension_semantics=("parallel", "arbitrary")),
    )(xf, Wq, mem_keys)


SC_NC, SC_NS = 2, 16
SC_NW = SC_NC * SC_NS
R_TOTAL = N * K
R_PER_W = R_TOTAL // SC_NW


def _sc_gather_body(values_hbm, idx_hbm, out_hbm, idx_v, rows_v, sem):
    wid = lax.axis_index("s") * SC_NC + lax.axis_index("c")
    base = wid * R_PER_W
    pltpu.sync_copy(idx_hbm.at[pl.ds(base, R_PER_W)], idx_v)
    pltpu.async_copy(values_hbm.at[idx_v], rows_v, sem).wait()
    pltpu.sync_copy(rows_v, out_hbm.at[pl.ds(base, R_PER_W)])


def _sc_gather(mem_values, idx_flat):
    mesh = plsc.VectorSubcoreMesh(core_axis_name="c", subcore_axis_name="s")
    return pl.kernel(
        _sc_gather_body,
        mesh=mesh,
        out_type=jax.ShapeDtypeStruct((R_TOTAL, D), jnp.float32),
        scratch_types=[
            pltpu.VMEM((R_PER_W,), jnp.int32),
            pltpu.VMEM((R_PER_W, D), jnp.float32),
            pltpu.SemaphoreType.DMA,
        ],
    )(mem_values, idx_flat)


def _epilogue_kernel(x_ref, vals_ref, g_ref, wout_ref, wg1_ref, bg1_ref,
                     wg2_ref, bg2_ref, out_ref):
    vals = vals_ref[...]
    mx = jnp.max(vals, axis=1, keepdims=True)
    e = jnp.exp(vals - mx)
    w = e / jnp.sum(e, axis=1, keepdims=True)

    row_id = lax.broadcasted_iota(jnp.int32, (K, D), 0)
    g = g_ref[...]
    r = jnp.zeros((TT, D), jnp.float32)
    for k in range(K):
        sel = jnp.where(row_id == k, 1.0, 0.0)
        wb = jnp.dot(w, sel, preferred_element_type=jnp.float32)
        r = r + wb * g[:, k, :]

    r = jnp.dot(r, wout_ref[...].T, preferred_element_type=jnp.float32)
    x = x_ref[...]
    gate_in = jnp.concatenate([x, r], axis=1)
    h = jnp.dot(gate_in, wg1_ref[...].T,
                preferred_element_type=jnp.float32) + bg1_ref[...]
    h = 0.5 * h * (1.0 + lax.erf(h * (1.0 / math.sqrt(2.0))))
    gate = jnp.dot(h, wg2_ref[...].T,
                   preferred_element_type=jnp.float32) + bg2_ref[...]
    gate = jax.nn.sigmoid(gate)
    out_ref[...] = x + gate * r


def _epilogue(xf, vals, gathered, Wout, Wg1, bg1, Wg2, bg2):
    H = Wg1.shape[0]
    return pl.pallas_call(
        _epilogue_kernel,
        grid=(NT,),
        in_specs=[
            pl.BlockSpec((TT, D), lambda t: (t, 0)),
            pl.BlockSpec((TT, K), lambda t: (t, 0)),
            pl.BlockSpec((TT, K, D), lambda t: (t, 0, 0)),
            pl.BlockSpec((D, D), lambda t: (0, 0)),
            pl.BlockSpec((H, 2 * D), lambda t: (0, 0)),
            pl.BlockSpec((1, H), lambda t: (0, 0)),
            pl.BlockSpec((D, H), lambda t: (0, 0)),
            pl.BlockSpec((1, D), lambda t: (0, 0)),
        ],
        out_specs=pl.BlockSpec((TT, D), lambda t: (t, 0)),
        out_shape=jax.ShapeDtypeStruct((N, D), jnp.float32),
        compiler_params=pltpu.CompilerParams(
            dimension_semantics=("parallel",)),
    )(xf, vals, gathered, Wout, Wg1, bg1.reshape(1, H),
      jnp.broadcast_to(Wg2, (D, H)), jnp.broadcast_to(bg2.reshape(1, 1),
                                                      (1, D)))


@jax.jit
def kernel(x, Wq, mem_keys, mem_values, Wout, Wg1, bg1, Wg2, bg2):
    xf = x.reshape(N, D)
    vals, idx = _topk(xf, Wq, mem_keys)
    gathered = _sc_gather(mem_values, idx.reshape(R_TOTAL))
    gathered = gathered.reshape(N, K, D)
    out = _epilogue(xf, vals, gathered, Wout, Wg1, bg1, Wg2, bg2)
    return out.reshape(B, T, D)


# speedup vs baseline: 40.0698x; 1.4010x over previous
import functools
import math

import jax
import jax.numpy as jnp
from jax import lax
from jax.experimental import pallas as pl
from jax.experimental.pallas import tpu as pltpu
from jax.experimental.pallas import tpu_sc as plsc

B, T, D, M, K = 2, 1024, 128, 32768, 8
N = B * T
TT = 256
TM = 2048
NT = N // TT
NM = M // TM
INV_SQRT_D = 1.0 / math.sqrt(D)
NEG_INF = float("-inf")


def _topk_kernel(x_ref, wq_ref, keys_ref, vals_ref, idx_ref, q_scr):
    m = pl.program_id(1)

    @pl.when(m == 0)
    def _init():
        q_scr[...] = jnp.dot(x_ref[...], wq_ref[...].T,
                             preferred_element_type=jnp.float32)
        vals_ref[...] = jnp.full((TT, K), NEG_INF, jnp.float32)
        idx_ref[...] = jnp.zeros((TT, K), jnp.int32)

    sim = jnp.dot(q_scr[...], keys_ref[...].T,
                  preferred_element_type=jnp.float32) * INV_SQRT_D

    iota = lax.broadcasted_iota(jnp.int32, (TT, TM), 1).astype(jnp.float32)
    tvals = []
    tidx = []
    s = sim
    for _ in range(K):
        mx = jnp.max(s, axis=1, keepdims=True)
        pos = jnp.min(jnp.where(s == mx, iota, float(TM)), axis=1,
                      keepdims=True)
        tvals.append(mx)
        tidx.append(pos + (m * TM).astype(jnp.float32))
        s = jnp.where(iota == pos, NEG_INF, s)

    cat_v = jnp.concatenate([vals_ref[...]] + tvals, axis=1)
    cat_i = jnp.concatenate([idx_ref[...].astype(jnp.float32)] + tidx,
                            axis=1)
    iota2 = lax.broadcasted_iota(jnp.int32, (TT, 2 * K), 1).astype(
        jnp.float32)
    nv = []
    ni = []
    for _ in range(K):
        mx = jnp.max(cat_v, axis=1, keepdims=True)
        pos = jnp.min(jnp.where(cat_v == mx, iota2, float(2 * K)), axis=1,
                      keepdims=True)
        sel = jnp.min(jnp.where(iota2 == pos, cat_i, float(M)), axis=1,
                      keepdims=True)
        nv.append(mx)
        ni.append(sel)
        cat_v = jnp.where(iota2 == pos, NEG_INF, cat_v)
    vals_ref[...] = jnp.concatenate(nv, axis=1)
    idx_ref[...] = jnp.concatenate(ni, axis=1).astype(jnp.int32)


def _topk(xf, Wq, mem_keys):
    return pl.pallas_call(
        _topk_kernel,
        grid=(NT, NM),
        in_specs=[
            pl.BlockSpec((TT, D), lambda t, m: (t, 0)),
            pl.BlockSpec((D, D), lambda t, m: (0, 0)),
            pl.BlockSpec((TM, D), lambda t, m: (m, 0)),
        ],
        out_specs=[
            pl.BlockSpec((TT, K), lambda t, m: (t, 0)),
            pl.BlockSpec((TT, K), lambda t, m: (t, 0)),
        ],
        out_shape=[
            jax.ShapeDtypeStruct((N, K), jnp.float32),
            jax.ShapeDtypeStruct((N, K), jnp.int32),
        ],
        scratch_shapes=[pltpu.VMEM((TT, D), jnp.float32)],
        compiler_params=pltpu.CompilerParams(
            dimension_semantics=("parallel", "arbitrary")),
    )(xf, Wq, mem_keys)


SC_NC, SC_NS = 2, 16
SC_NW = SC_NC * SC_NS
R_TOTAL = N * K
R_PER_W = R_TOTAL // SC_NW


def _sc_gather_body(values_hbm, idx_hbm, out_hbm, idx_v, rows_v, sem):
    wid = lax.axis_index("s") * SC_NC + lax.axis_index("c")
    base = wid * R_PER_W
    pltpu.sync_copy(idx_hbm.at[pl.ds(base, R_PER_W)], idx_v)
    pltpu.async_copy(values_hbm.at[idx_v], rows_v, sem).wait()
    pltpu.sync_copy(rows_v, out_hbm.at[pl.ds(base, R_PER_W)])


def _sc_gather(mem_values, idx_flat):
    mesh = plsc.VectorSubcoreMesh(core_axis_name="c", subcore_axis_name="s")
    return pl.kernel(
        _sc_gather_body,
        mesh=mesh,
        out_type=jax.ShapeDtypeStruct((R_TOTAL, D), jnp.float32),
        scratch_types=[
            pltpu.VMEM((R_PER_W,), jnp.int32),
            pltpu.VMEM((R_PER_W, D), jnp.float32),
            pltpu.SemaphoreType.DMA,
        ],
    )(mem_values, idx_flat)


def _epilogue_kernel(x_ref, vals_ref, g_ref, wout_ref, wg1_ref, bg1_ref,
                     wg2_ref, bg2_ref, out_ref):
    vals = vals_ref[...]
    mx = jnp.max(vals, axis=1, keepdims=True)
    e = jnp.exp(vals - mx)
    w = e / jnp.sum(e, axis=1, keepdims=True)

    row_id = lax.broadcasted_iota(jnp.int32, (K, D), 0)
    g = g_ref[...]
    r = jnp.zeros((TT, D), jnp.float32)
    for k in range(K):
        sel = jnp.where(row_id == k, 1.0, 0.0)
        wb = jnp.dot(w, sel, preferred_element_type=jnp.float32)
        r = r + wb * g[:, k, :]

    r = jnp.dot(r, wout_ref[...].T, preferred_element_type=jnp.float32)
    x = x_ref[...]
    gate_in = jnp.concatenate([x, r], axis=1)
    h = jnp.dot(gate_in, wg1_ref[...].T,
                preferred_element_type=jnp.float32) + bg1_ref[...]
    h = 0.5 * h * (1.0 + lax.erf(h * (1.0 / math.sqrt(2.0))))
    gate = jnp.dot(h, wg2_ref[...].T,
                   preferred_element_type=jnp.float32) + bg2_ref[...]
    gate = jax.nn.sigmoid(gate)
    out_ref[...] = x + gate * r


def _epilogue(xf, vals, gathered, Wout, Wg1, bg1, Wg2, bg2):
    H = Wg1.shape[0]
    return pl.pallas_call(
        _epilogue_kernel,
        grid=(NT,),
        in_specs=[
            pl.BlockSpec((TT, D), lambda t: (t, 0)),
            pl.BlockSpec((TT, K), lambda t: (t, 0)),
            pl.BlockSpec((TT, K, D), lambda t: (t, 0, 0)),
            pl.BlockSpec((D, D), lambda t: (0, 0)),
            pl.BlockSpec((H, 2 * D), lambda t: (0, 0)),
            pl.BlockSpec((1, H), lambda t: (0, 0)),
            pl.BlockSpec((D, H), lambda t: (0, 0)),
            pl.BlockSpec((1, D), lambda t: (0, 0)),
        ],
        out_specs=pl.BlockSpec((TT, D), lambda t: (t, 0)),
        out_shape=jax.ShapeDtypeStruct((N, D), jnp.float32),
        compiler_params=pltpu.CompilerParams(
            dimension_semantics=("parallel",)),
    )(xf, vals, gathered, Wout, Wg1, bg1.reshape(1, H),
      jnp.broadcast_to(Wg2, (D, H)), jnp.broadcast_to(bg2.reshape(1, 1),
                                                      (1, D)))


@jax.jit
def kernel(x, Wq, mem_keys, mem_values, Wout, Wg1, bg1, Wg2, bg2):
    xf = x.reshape(N, D)
    vals, idx = _topk(xf, Wq, mem_keys)
    gathered = _sc_gather(mem_values, idx.reshape(R_TOTAL))
    gathered = gathered.reshape(N, K, D)
    out = _epilogue(xf, vals, gathered, Wout, Wg1, bg1, Wg2, bg2)
    return out.reshape(B, T, D)


# speedup vs baseline: 46.6628x; 1.1645x over previous
import functools
import math

import jax
import jax.numpy as jnp
from jax import lax
from jax.experimental import pallas as pl
from jax.experimental.pallas import tpu as pltpu
from jax.experimental.pallas import tpu_sc as plsc

B, T, D, M, K = 2, 1024, 128, 32768, 8
N = B * T
TT = 256
TM = 4096
NT = N // TT
NM = M // TM
INV_SQRT_D = 1.0 / math.sqrt(D)
NEG_INF = float("-inf")


def _topk_kernel(x_ref, wq_ref, keys_ref, vals_ref, idx_ref, q_scr):
    m = pl.program_id(1)

    @pl.when(m == 0)
    def _init():
        q_scr[...] = jnp.dot(x_ref[...], wq_ref[...].T,
                             preferred_element_type=jnp.float32)
        vals_ref[...] = jnp.full((TT, K), NEG_INF, jnp.float32)
        idx_ref[...] = jnp.zeros((TT, K), jnp.int32)

    sim = jnp.dot(q_scr[...], keys_ref[...].T,
                  preferred_element_type=jnp.float32) * INV_SQRT_D

    iota = lax.broadcasted_iota(jnp.int32, (TT, TM), 1).astype(jnp.float32)
    tvals = []
    tidx = []
    s = sim
    for _ in range(K):
        mx = jnp.max(s, axis=1, keepdims=True)
        pos = jnp.min(jnp.where(s == mx, iota, float(TM)), axis=1,
                      keepdims=True)
        tvals.append(mx)
        tidx.append(pos + (m * TM).astype(jnp.float32))
        s = jnp.where(iota == pos, NEG_INF, s)

    cat_v = jnp.concatenate([vals_ref[...]] + tvals, axis=1)
    cat_i = jnp.concatenate([idx_ref[...].astype(jnp.float32)] + tidx,
                            axis=1)
    iota2 = lax.broadcasted_iota(jnp.int32, (TT, 2 * K), 1).astype(
        jnp.float32)
    nv = []
    ni = []
    for _ in range(K):
        mx = jnp.max(cat_v, axis=1, keepdims=True)
        pos = jnp.min(jnp.where(cat_v == mx, iota2, float(2 * K)), axis=1,
                      keepdims=True)
        sel = jnp.min(jnp.where(iota2 == pos, cat_i, float(M)), axis=1,
                      keepdims=True)
        nv.append(mx)
        ni.append(sel)
        cat_v = jnp.where(iota2 == pos, NEG_INF, cat_v)
    vals_ref[...] = jnp.concatenate(nv, axis=1)
    idx_ref[...] = jnp.concatenate(ni, axis=1).astype(jnp.int32)


def _topk(xf, Wq, mem_keys):
    return pl.pallas_call(
        _topk_kernel,
        grid=(NT, NM),
        in_specs=[
            pl.BlockSpec((TT, D), lambda t, m: (t, 0)),
            pl.BlockSpec((D, D), lambda t, m: (0, 0)),
            pl.BlockSpec((TM, D), lambda t, m: (m, 0)),
        ],
        out_specs=[
            pl.BlockSpec((TT, K), lambda t, m: (t, 0)),
            pl.BlockSpec((TT, K), lambda t, m: (t, 0)),
        ],
        out_shape=[
            jax.ShapeDtypeStruct((N, K), jnp.float32),
            jax.ShapeDtypeStruct((N, K), jnp.int32),
        ],
        scratch_shapes=[pltpu.VMEM((TT, D), jnp.float32)],
        compiler_params=pltpu.CompilerParams(
            dimension_semantics=("parallel", "arbitrary")),
    )(xf, Wq, mem_keys)


SC_NC, SC_NS = 2, 16
SC_NW = SC_NC * SC_NS
R_TOTAL = N * K
R_PER_W = R_TOTAL // SC_NW


def _sc_gather_body(values_hbm, idx_hbm, out_hbm, idx_v, rows_v, sem):
    wid = lax.axis_index("s") * SC_NC + lax.axis_index("c")
    base = wid * R_PER_W
    pltpu.sync_copy(idx_hbm.at[pl.ds(base, R_PER_W)], idx_v)
    pltpu.async_copy(values_hbm.at[idx_v], rows_v, sem).wait()
    pltpu.sync_copy(rows_v, out_hbm.at[pl.ds(base, R_PER_W)])


def _sc_gather(mem_values, idx_flat):
    mesh = plsc.VectorSubcoreMesh(core_axis_name="c", subcore_axis_name="s")
    return pl.kernel(
        _sc_gather_body,
        mesh=mesh,
        out_type=jax.ShapeDtypeStruct((R_TOTAL, D), jnp.float32),
        scratch_types=[
            pltpu.VMEM((R_PER_W,), jnp.int32),
            pltpu.VMEM((R_PER_W, D), jnp.float32),
            pltpu.SemaphoreType.DMA,
        ],
    )(mem_values, idx_flat)


def _epilogue_kernel(x_ref, vals_ref, g_ref, wout_ref, wg1_ref, bg1_ref,
                     wg2_ref, bg2_ref, out_ref):
    vals = vals_ref[...]
    mx = jnp.max(vals, axis=1, keepdims=True)
    e = jnp.exp(vals - mx)
    w = e / jnp.sum(e, axis=1, keepdims=True)

    row_id = lax.broadcasted_iota(jnp.int32, (K, D), 0)
    g = g_ref[...]
    r = jnp.zeros((TT, D), jnp.float32)
    for k in range(K):
        sel = jnp.where(row_id == k, 1.0, 0.0)
        wb = jnp.dot(w, sel, preferred_element_type=jnp.float32)
        r = r + wb * g[:, k, :]

    r = jnp.dot(r, wout_ref[...].T, preferred_element_type=jnp.float32)
    x = x_ref[...]
    gate_in = jnp.concatenate([x, r], axis=1)
    h = jnp.dot(gate_in, wg1_ref[...].T,
                preferred_element_type=jnp.float32) + bg1_ref[...]
    h = 0.5 * h * (1.0 + lax.erf(h * (1.0 / math.sqrt(2.0))))
    gate = jnp.dot(h, wg2_ref[...].T,
                   preferred_element_type=jnp.float32) + bg2_ref[...]
    gate = jax.nn.sigmoid(gate)
    out_ref[...] = x + gate * r


def _epilogue(xf, vals, gathered, Wout, Wg1, bg1, Wg2, bg2):
    H = Wg1.shape[0]
    return pl.pallas_call(
        _epilogue_kernel,
        grid=(NT,),
        in_specs=[
            pl.BlockSpec((TT, D), lambda t: (t, 0)),
            pl.BlockSpec((TT, K), lambda t: (t, 0)),
            pl.BlockSpec((TT, K, D), lambda t: (t, 0, 0)),
            pl.BlockSpec((D, D), lambda t: (0, 0)),
            pl.BlockSpec((H, 2 * D), lambda t: (0, 0)),
            pl.BlockSpec((1, H), lambda t: (0, 0)),
            pl.BlockSpec((D, H), lambda t: (0, 0)),
            pl.BlockSpec((1, D), lambda t: (0, 0)),
        ],
        out_specs=pl.BlockSpec((TT, D), lambda t: (t, 0)),
        out_shape=jax.ShapeDtypeStruct((N, D), jnp.float32),
        compiler_params=pltpu.CompilerParams(
            dimension_semantics=("parallel",)),
    )(xf, vals, gathered, Wout, Wg1, bg1.reshape(1, H),
      jnp.broadcast_to(Wg2, (D, H)), jnp.broadcast_to(bg2.reshape(1, 1),
                                                      (1, D)))


@jax.jit
def kernel(x, Wq, mem_keys, mem_values, Wout, Wg1, bg1, Wg2, bg2):
    xf = x.reshape(N, D)
    vals, idx = _topk(xf, Wq, mem_keys)
    gathered = _sc_gather(mem_values, idx.reshape(R_TOTAL))
    gathered = gathered.reshape(N, K, D)
    out = _epilogue(xf, vals, gathered, Wout, Wg1, bg1, Wg2, bg2)
    return out.reshape(B, T, D)


# speedup vs baseline: 50.0656x; 1.0729x over previous
import functools
import math

import jax
import jax.numpy as jnp
from jax import lax
from jax.experimental import pallas as pl
from jax.experimental.pallas import tpu as pltpu
from jax.experimental.pallas import tpu_sc as plsc

B, T, D, M, K = 2, 1024, 128, 32768, 8
N = B * T
TT = 512
TM = 4096
NT = N // TT
NM = M // TM
INV_SQRT_D = 1.0 / math.sqrt(D)
NEG_INF = float("-inf")


def _topk_kernel(x_ref, wq_ref, keys_ref, vals_ref, idx_ref, q_scr):
    m = pl.program_id(1)

    @pl.when(m == 0)
    def _init():
        q_scr[...] = jnp.dot(x_ref[...], wq_ref[...].T,
                             preferred_element_type=jnp.float32)
        vals_ref[...] = jnp.full((TT, K), NEG_INF, jnp.float32)
        idx_ref[...] = jnp.zeros((TT, K), jnp.int32)

    sim = jnp.dot(q_scr[...], keys_ref[...].T,
                  preferred_element_type=jnp.float32) * INV_SQRT_D

    iota = lax.broadcasted_iota(jnp.int32, (TT, TM), 1).astype(jnp.float32)
    tvals = []
    tidx = []
    s = sim
    for _ in range(K):
        mx = jnp.max(s, axis=1, keepdims=True)
        pos = jnp.min(jnp.where(s == mx, iota, float(TM)), axis=1,
                      keepdims=True)
        tvals.append(mx)
        tidx.append(pos + (m * TM).astype(jnp.float32))
        s = jnp.where(iota == pos, NEG_INF, s)

    cat_v = jnp.concatenate([vals_ref[...]] + tvals, axis=1)
    cat_i = jnp.concatenate([idx_ref[...].astype(jnp.float32)] + tidx,
                            axis=1)
    iota2 = lax.broadcasted_iota(jnp.int32, (TT, 2 * K), 1).astype(
        jnp.float32)
    nv = []
    ni = []
    for _ in range(K):
        mx = jnp.max(cat_v, axis=1, keepdims=True)
        pos = jnp.min(jnp.where(cat_v == mx, iota2, float(2 * K)), axis=1,
                      keepdims=True)
        sel = jnp.min(jnp.where(iota2 == pos, cat_i, float(M)), axis=1,
                      keepdims=True)
        nv.append(mx)
        ni.append(sel)
        cat_v = jnp.where(iota2 == pos, NEG_INF, cat_v)
    vals_ref[...] = jnp.concatenate(nv, axis=1)
    idx_ref[...] = jnp.concatenate(ni, axis=1).astype(jnp.int32)


def _topk(xf, Wq, mem_keys):
    return pl.pallas_call(
        _topk_kernel,
        grid=(NT, NM),
        in_specs=[
            pl.BlockSpec((TT, D), lambda t, m: (t, 0)),
            pl.BlockSpec((D, D), lambda t, m: (0, 0)),
            pl.BlockSpec((TM, D), lambda t, m: (m, 0)),
        ],
        out_specs=[
            pl.BlockSpec((TT, K), lambda t, m: (t, 0)),
            pl.BlockSpec((TT, K), lambda t, m: (t, 0)),
        ],
        out_shape=[
            jax.ShapeDtypeStruct((N, K), jnp.float32),
            jax.ShapeDtypeStruct((N, K), jnp.int32),
        ],
        scratch_shapes=[pltpu.VMEM((TT, D), jnp.float32)],
        compiler_params=pltpu.CompilerParams(
            dimension_semantics=("parallel", "arbitrary")),
    )(xf, Wq, mem_keys)


SC_NC, SC_NS = 2, 16
SC_NW = SC_NC * SC_NS
R_TOTAL = N * K
R_PER_W = R_TOTAL // SC_NW


def _sc_gather_body(values_hbm, idx_hbm, out_hbm, idx_v, rows_v, sem):
    wid = lax.axis_index("s") * SC_NC + lax.axis_index("c")
    base = wid * R_PER_W
    pltpu.sync_copy(idx_hbm.at[pl.ds(base, R_PER_W)], idx_v)
    pltpu.async_copy(values_hbm.at[idx_v], rows_v, sem).wait()
    pltpu.sync_copy(rows_v, out_hbm.at[pl.ds(base, R_PER_W)])


def _sc_gather(mem_values, idx_flat):
    mesh = plsc.VectorSubcoreMesh(core_axis_name="c", subcore_axis_name="s")
    return pl.kernel(
        _sc_gather_body,
        mesh=mesh,
        out_type=jax.ShapeDtypeStruct((R_TOTAL, D), jnp.float32),
        scratch_types=[
            pltpu.VMEM((R_PER_W,), jnp.int32),
            pltpu.VMEM((R_PER_W, D), jnp.float32),
            pltpu.SemaphoreType.DMA,
        ],
    )(mem_values, idx_flat)


def _epilogue_kernel(x_ref, vals_ref, g_ref, wout_ref, wg1_ref, bg1_ref,
                     wg2_ref, bg2_ref, out_ref):
    vals = vals_ref[...]
    mx = jnp.max(vals, axis=1, keepdims=True)
    e = jnp.exp(vals - mx)
    w = e / jnp.sum(e, axis=1, keepdims=True)

    row_id = lax.broadcasted_iota(jnp.int32, (K, D), 0)
    g = g_ref[...]
    r = jnp.zeros((TT, D), jnp.float32)
    for k in range(K):
        sel = jnp.where(row_id == k, 1.0, 0.0)
        wb = jnp.dot(w, sel, preferred_element_type=jnp.float32)
        r = r + wb * g[:, k, :]

    r = jnp.dot(r, wout_ref[...].T, preferred_element_type=jnp.float32)
    x = x_ref[...]
    gate_in = jnp.concatenate([x, r], axis=1)
    h = jnp.dot(gate_in, wg1_ref[...].T,
                preferred_element_type=jnp.float32) + bg1_ref[...]
    h = 0.5 * h * (1.0 + lax.erf(h * (1.0 / math.sqrt(2.0))))
    gate = jnp.dot(h, wg2_ref[...].T,
                   preferred_element_type=jnp.float32) + bg2_ref[...]
    gate = jax.nn.sigmoid(gate)
    out_ref[...] = x + gate * r


def _epilogue(xf, vals, gathered, Wout, Wg1, bg1, Wg2, bg2):
    H = Wg1.shape[0]
    return pl.pallas_call(
        _epilogue_kernel,
        grid=(NT,),
        in_specs=[
            pl.BlockSpec((TT, D), lambda t: (t, 0)),
            pl.BlockSpec((TT, K), lambda t: (t, 0)),
            pl.BlockSpec((TT, K, D), lambda t: (t, 0, 0)),
            pl.BlockSpec((D, D), lambda t: (0, 0)),
            pl.BlockSpec((H, 2 * D), lambda t: (0, 0)),
            pl.BlockSpec((1, H), lambda t: (0, 0)),
            pl.BlockSpec((D, H), lambda t: (0, 0)),
            pl.BlockSpec((1, D), lambda t: (0, 0)),
        ],
        out_specs=pl.BlockSpec((TT, D), lambda t: (t, 0)),
        out_shape=jax.ShapeDtypeStruct((N, D), jnp.float32),
        compiler_params=pltpu.CompilerParams(
            dimension_semantics=("parallel",)),
    )(xf, vals, gathered, Wout, Wg1, bg1.reshape(1, H),
      jnp.broadcast_to(Wg2, (D, H)), jnp.broadcast_to(bg2.reshape(1, 1),
                                                      (1, D)))


@jax.jit
def kernel(x, Wq, mem_keys, mem_values, Wout, Wg1, bg1, Wg2, bg2):
    xf = x.reshape(N, D)
    vals, idx = _topk(xf, Wq, mem_keys)
    gathered = _sc_gather(mem_values, idx.reshape(R_TOTAL))
    gathered = gathered.reshape(N, K, D)
    out = _epilogue(xf, vals, gathered, Wout, Wg1, bg1, Wg2, bg2)
    return out.reshape(B, T, D)
